# baseline (speedup 1.0000x reference)
import jax
import jax.numpy as jnp
from jax import lax
from jax.experimental import pallas as pl
from jax.experimental.pallas import tpu as pltpu
from jax.experimental.pallas import tpu_sc as plsc

_B = 16384
_LANES = 16
_TILES = 16
_ROWS_PER_TILE = _B // _TILES
_CHUNK = 256


def _sc_body(ig_hbm, sc_hbm, sib_hbm, h_hbm, igm_hbm, scm_hbm, cnt_hbm,
             opn_hbm, ho_hbm,
             ig_v, sc_v, igm_v, scm_v, sib_v, cnt_v, opn_v, h_v, tot_smem):
    cid = lax.axis_index("c")
    sid = lax.axis_index("s")

    @pl.when(cid == 1)
    def _core1():
        hbase = sid * _ROWS_PER_TILE
        for c in range(_ROWS_PER_TILE // _CHUNK):
            hb = hbase + c * _CHUNK
            pltpu.sync_copy(h_hbm.at[pl.ds(hb, _CHUNK)], h_v)
            pltpu.sync_copy(h_v, ho_hbm.at[pl.ds(hb, _CHUNK)])

    @pl.when(cid == 0)
    def _core0():
        base = sid * _ROWS_PER_TILE
        pltpu.sync_copy(sib_hbm, sib_v)
        sib16 = sib_v[...]
        iota16 = lax.iota(jnp.int32, 16)
        acc = jnp.zeros((_LANES,), jnp.int32)

        for c in range(_ROWS_PER_TILE // _CHUNK):
            cb = base + c * _CHUNK
            pltpu.sync_copy(ig_hbm.at[pl.ds(cb, _CHUNK)], ig_v)
            pltpu.sync_copy(sc_hbm.at[pl.ds(cb, _CHUNK)], sc_v)

            def step(j, acc, c=c):
                row = j * _LANES + iota16
                dst = c * _CHUNK + j * _LANES
                igm_v[pl.ds(dst, _LANES)] = plsc.load_gather(
                    ig_v, [row, sib16])
                scx = plsc.load_gather(sc_v, [row, sib16])
                scm_v[pl.ds(dst, _LANES)] = scx
                return acc + scx

            acc = lax.fori_loop(0, _CHUNK // _LANES, step, acc)

        pltpu.sync_copy(igm_v, igm_hbm.at[pl.ds(base, _ROWS_PER_TILE)])
        pltpu.sync_copy(scm_v, scm_hbm.at[pl.ds(base, _ROWS_PER_TILE)])

        my_sum = jnp.sum(acc)

        @pl.when(sid == 0)
        def _init():
            tot_smem[0] = jnp.int32(0)

        plsc.subcore_barrier()
        plsc.fetch_and_add(tot_smem.at[0], my_sum, subcore_id=0)
        plsc.subcore_barrier()

        @pl.when(sid == 0)
        def _finalize():
            total = tot_smem[0].astype(jnp.float32)
            cnt_v[...] = jnp.broadcast_to(total, (_LANES,))
            opn_v[...] = jnp.broadcast_to(
                (total > 0.0).astype(jnp.int32), (_LANES,))
            pltpu.sync_copy(cnt_v, cnt_hbm)
            pltpu.sync_copy(opn_v, opn_hbm)


@jax.jit
def _sc_call(parent_ig_matrix, parent_sc_matrix, sib16, h_input):
    mesh = plsc.VectorSubcoreMesh(core_axis_name="c", subcore_axis_name="s")
    run = pl.kernel(
        _sc_body,
        out_type=[
            jax.ShapeDtypeStruct((_B,), jnp.int32),
            jax.ShapeDtypeStruct((_B,), jnp.int32),
            jax.ShapeDtypeStruct((_LANES,), jnp.float32),
            jax.ShapeDtypeStruct((_LANES,), jnp.int32),
            jax.ShapeDtypeStruct((_B, 128), jnp.float32),
        ],
        mesh=mesh,
        scratch_types=[
            pltpu.VMEM((_CHUNK, 2), jnp.int32),
            pltpu.VMEM((_CHUNK, 2), jnp.int32),
            pltpu.VMEM((_ROWS_PER_TILE,), jnp.int32),
            pltpu.VMEM((_ROWS_PER_TILE,), jnp.int32),
            pltpu.VMEM((_LANES,), jnp.int32),
            pltpu.VMEM((_LANES,), jnp.float32),
            pltpu.VMEM((_LANES,), jnp.int32),
            pltpu.VMEM((_CHUNK, 128), jnp.float32),
            pltpu.SMEM((1,), jnp.int32),
        ],
        compiler_params=pltpu.CompilerParams(
            needs_layout_passes=False,
            use_tc_tiling_on_sc=True,
            skip_device_barrier=True,
        ),
        name="cign_masking_sc",
    )
    return run(parent_ig_matrix, parent_sc_matrix, sib16, h_input)


_N_BLK = 16
_RB = _B // _N_BLK


def _tc_body(f_hbm, fo_hbm, sem):
    copy = pltpu.make_async_copy(f_hbm, fo_hbm, sem)
    copy.start()
    copy.wait()


@jax.jit
def _tc_call(f_input):
    return pl.pallas_call(
        _tc_body,
        in_specs=[pl.BlockSpec(memory_space=pltpu.MemorySpace.HBM)],
        out_specs=pl.BlockSpec(memory_space=pltpu.MemorySpace.HBM),
        out_shape=jax.ShapeDtypeStruct((_B, 1024), jnp.float32),
        scratch_shapes=[pltpu.SemaphoreType.DMA],
    )(f_input)


def kernel(f_input, h_input, parent_ig_matrix, parent_sc_matrix, sibling_index):
    sib16 = jnp.full((_LANES,), sibling_index, dtype=jnp.int32)
    igm, scm, cnt, opn, h_out = _sc_call(
        parent_ig_matrix, parent_sc_matrix, sib16, h_input)
    f_out = _tc_call(f_input)
    sample_count = cnt[0]
    is_node_open = opn[0].astype(jnp.bool_)
    return (f_out, h_out, igm, scm, sample_count, is_node_open)


# speedup vs baseline: 24.0559x; 24.0559x over previous
import jax
import jax.numpy as jnp
from jax import lax
from jax.experimental import pallas as pl
from jax.experimental.pallas import tpu as pltpu
from jax.experimental.pallas import tpu_sc as plsc

_B = 16384
_LANES = 16
_TILES = 16
_ROWS_PER_TILE = _B // _TILES
_CHUNK = 256


def _sc_body(ig_hbm, sc_hbm, sib_hbm, h_hbm, igm_hbm, scm_hbm, cnt_hbm,
             opn_hbm, ho_hbm,
             ig_v, sc_v, igm_v, scm_v, sib_v, cnt_v, opn_v, h_v, tot_smem):
    cid = lax.axis_index("c")
    sid = lax.axis_index("s")

    @pl.when(cid == 1)
    def _core1():
        hbase = sid * _ROWS_PER_TILE
        for c in range(_ROWS_PER_TILE // _CHUNK):
            hb = hbase + c * _CHUNK
            pltpu.sync_copy(h_hbm.at[pl.ds(hb, _CHUNK)], h_v)
            pltpu.sync_copy(h_v, ho_hbm.at[pl.ds(hb, _CHUNK)])

    @pl.when(cid == 0)
    def _core0():
        base = sid * _ROWS_PER_TILE
        pltpu.sync_copy(sib_hbm, sib_v)
        sib16 = sib_v[...]
        iota16 = lax.iota(jnp.int32, 16)
        acc = jnp.zeros((_LANES,), jnp.int32)

        for c in range(_ROWS_PER_TILE // _CHUNK):
            cb = base + c * _CHUNK
            pltpu.sync_copy(ig_hbm.at[pl.ds(cb, _CHUNK)], ig_v)
            pltpu.sync_copy(sc_hbm.at[pl.ds(cb, _CHUNK)], sc_v)

            def step(j, acc, c=c):
                row = j * _LANES + iota16
                dst = c * _CHUNK + j * _LANES
                igm_v[pl.ds(dst, _LANES)] = plsc.load_gather(
                    ig_v, [row, sib16])
                scx = plsc.load_gather(sc_v, [row, sib16])
                scm_v[pl.ds(dst, _LANES)] = scx
                return acc + scx

            acc = lax.fori_loop(0, _CHUNK // _LANES, step, acc)

        pltpu.sync_copy(igm_v, igm_hbm.at[pl.ds(base, _ROWS_PER_TILE)])
        pltpu.sync_copy(scm_v, scm_hbm.at[pl.ds(base, _ROWS_PER_TILE)])

        my_sum = jnp.sum(acc)

        @pl.when(sid == 0)
        def _init():
            tot_smem[0] = jnp.int32(0)

        plsc.subcore_barrier()
        plsc.fetch_and_add(tot_smem.at[0], my_sum, subcore_id=0)
        plsc.subcore_barrier()

        @pl.when(sid == 0)
        def _finalize():
            total = tot_smem[0].astype(jnp.float32)
            cnt_v[...] = jnp.broadcast_to(total, (_LANES,))
            opn_v[...] = jnp.broadcast_to(
                (total > 0.0).astype(jnp.int32), (_LANES,))
            pltpu.sync_copy(cnt_v, cnt_hbm)
            pltpu.sync_copy(opn_v, opn_hbm)


@jax.jit
def _sc_call(parent_ig_matrix, parent_sc_matrix, sib16, h_input):
    mesh = plsc.VectorSubcoreMesh(core_axis_name="c", subcore_axis_name="s")
    run = pl.kernel(
        _sc_body,
        out_type=[
            jax.ShapeDtypeStruct((_B,), jnp.int32),
            jax.ShapeDtypeStruct((_B,), jnp.int32),
            jax.ShapeDtypeStruct((_LANES,), jnp.float32),
            jax.ShapeDtypeStruct((_LANES,), jnp.int32),
            jax.ShapeDtypeStruct((_B, 128), jnp.float32),
        ],
        mesh=mesh,
        scratch_types=[
            pltpu.VMEM((_CHUNK, 2), jnp.int32),
            pltpu.VMEM((_CHUNK, 2), jnp.int32),
            pltpu.VMEM((_ROWS_PER_TILE,), jnp.int32),
            pltpu.VMEM((_ROWS_PER_TILE,), jnp.int32),
            pltpu.VMEM((_LANES,), jnp.int32),
            pltpu.VMEM((_LANES,), jnp.float32),
            pltpu.VMEM((_LANES,), jnp.int32),
            pltpu.VMEM((_CHUNK, 128), jnp.float32),
            pltpu.SMEM((1,), jnp.int32),
        ],
        compiler_params=pltpu.CompilerParams(
            needs_layout_passes=False,
            use_tc_tiling_on_sc=True,
            skip_device_barrier=True,
        ),
        name="cign_masking_sc",
    )
    return run(parent_ig_matrix, parent_sc_matrix, sib16, h_input)


_N_BLK = 32
_RB = _B // _N_BLK


def _tc_body(f_ref, fo_ref):
    fo_ref[...] = f_ref[...]


@jax.jit
def _tc_call(f_input):
    return pl.pallas_call(
        _tc_body,
        grid=(_N_BLK,),
        in_specs=[pl.BlockSpec((_RB, 1024), lambda i: (i, 0))],
        out_specs=pl.BlockSpec((_RB, 1024), lambda i: (i, 0)),
        out_shape=jax.ShapeDtypeStruct((_B, 1024), jnp.float32),
    )(f_input)


def kernel(f_input, h_input, parent_ig_matrix, parent_sc_matrix, sibling_index):
    sib16 = jnp.full((_LANES,), sibling_index, dtype=jnp.int32)
    igm, scm, cnt, opn, h_out = _sc_call(
        parent_ig_matrix, parent_sc_matrix, sib16, h_input)
    f_out = _tc_call(f_input)
    sample_count = cnt[0]
    is_node_open = opn[0].astype(jnp.bool_)
    return (f_out, h_out, igm, scm, sample_count, is_node_open)


# speedup vs baseline: 24.3586x; 1.0126x over previous
import jax
import jax.numpy as jnp
from jax import lax
from jax.experimental import pallas as pl
from jax.experimental.pallas import tpu as pltpu
from jax.experimental.pallas import tpu_sc as plsc

_B = 16384
_LANES = 16
_TILES = 16
_ROWS_PER_TILE = _B // _TILES
_CHUNK = 256


def _sc_body(ig_hbm, sc_hbm, sib_hbm, igm_hbm, scm_hbm, cnt_hbm, opn_hbm,
             ig_v, sc_v, igm_v, scm_v, sib_v, cnt_v, opn_v, tot_smem):
    cid = lax.axis_index("c")
    sid = lax.axis_index("s")

    @pl.when(cid == 0)
    def _core0():
        base = sid * _ROWS_PER_TILE
        pltpu.sync_copy(sib_hbm, sib_v)
        sib16 = sib_v[...]
        iota16 = lax.iota(jnp.int32, 16)
        acc = jnp.zeros((_LANES,), jnp.int32)

        for c in range(_ROWS_PER_TILE // _CHUNK):
            cb = base + c * _CHUNK
            pltpu.sync_copy(ig_hbm.at[pl.ds(cb, _CHUNK)], ig_v)
            pltpu.sync_copy(sc_hbm.at[pl.ds(cb, _CHUNK)], sc_v)

            def step(j, acc, c=c):
                row = j * _LANES + iota16
                dst = c * _CHUNK + j * _LANES
                igm_v[pl.ds(dst, _LANES)] = plsc.load_gather(
                    ig_v, [row, sib16])
                scx = plsc.load_gather(sc_v, [row, sib16])
                scm_v[pl.ds(dst, _LANES)] = scx
                return acc + scx

            acc = lax.fori_loop(0, _CHUNK // _LANES, step, acc)

        pltpu.sync_copy(igm_v, igm_hbm.at[pl.ds(base, _ROWS_PER_TILE)])
        pltpu.sync_copy(scm_v, scm_hbm.at[pl.ds(base, _ROWS_PER_TILE)])

        my_sum = jnp.sum(acc)

        @pl.when(sid == 0)
        def _init():
            tot_smem[0] = jnp.int32(0)

        plsc.subcore_barrier()
        plsc.fetch_and_add(tot_smem.at[0], my_sum, subcore_id=0)
        plsc.subcore_barrier()

        @pl.when(sid == 0)
        def _finalize():
            total = tot_smem[0].astype(jnp.float32)
            cnt_v[...] = jnp.broadcast_to(total, (_LANES,))
            opn_v[...] = jnp.broadcast_to(
                (total > 0.0).astype(jnp.int32), (_LANES,))
            pltpu.sync_copy(cnt_v, cnt_hbm)
            pltpu.sync_copy(opn_v, opn_hbm)


@jax.jit
def _sc_call(parent_ig_matrix, parent_sc_matrix, sib16):
    mesh = plsc.VectorSubcoreMesh(core_axis_name="c", subcore_axis_name="s")
    run = pl.kernel(
        _sc_body,
        out_type=[
            jax.ShapeDtypeStruct((_B,), jnp.int32),
            jax.ShapeDtypeStruct((_B,), jnp.int32),
            jax.ShapeDtypeStruct((_LANES,), jnp.float32),
            jax.ShapeDtypeStruct((_LANES,), jnp.int32),
        ],
        mesh=mesh,
        scratch_types=[
            pltpu.VMEM((_CHUNK, 2), jnp.int32),
            pltpu.VMEM((_CHUNK, 2), jnp.int32),
            pltpu.VMEM((_ROWS_PER_TILE,), jnp.int32),
            pltpu.VMEM((_ROWS_PER_TILE,), jnp.int32),
            pltpu.VMEM((_LANES,), jnp.int32),
            pltpu.VMEM((_LANES,), jnp.float32),
            pltpu.VMEM((_LANES,), jnp.int32),
            pltpu.SMEM((1,), jnp.int32),
        ],
        compiler_params=pltpu.CompilerParams(
            needs_layout_passes=False,
            use_tc_tiling_on_sc=True,
            skip_device_barrier=True,
        ),
        name="cign_masking_sc",
    )
    return run(parent_ig_matrix, parent_sc_matrix, sib16)


_N_BLK = 16
_RB = _B // _N_BLK


def _tc_f_body(f_ref, fo_ref):
    fo_ref[...] = f_ref[...]


@jax.jit
def _tc_f_call(f_input):
    return pl.pallas_call(
        _tc_f_body,
        grid=(_N_BLK,),
        in_specs=[pl.BlockSpec((_RB, 1024), lambda i: (i, 0))],
        out_specs=pl.BlockSpec((_RB, 1024), lambda i: (i, 0)),
        out_shape=jax.ShapeDtypeStruct((_B, 1024), jnp.float32),
    )(f_input)


def _tc_h_body(h_ref, dep_ref, ho_ref):
    del dep_ref
    ho_ref[...] = h_ref[...]


@jax.jit
def _tc_h_call(h_input, dep):
    return pl.pallas_call(
        _tc_h_body,
        grid=(4,),
        in_specs=[
            pl.BlockSpec((_B // 4, 128), lambda i: (i, 0)),
            pl.BlockSpec(memory_space=pltpu.MemorySpace.HBM),
        ],
        out_specs=pl.BlockSpec((_B // 4, 128), lambda i: (i, 0)),
        out_shape=jax.ShapeDtypeStruct((_B, 128), jnp.float32),
    )(h_input, dep)


def kernel(f_input, h_input, parent_ig_matrix, parent_sc_matrix, sibling_index):
    sib16 = jnp.full((_LANES,), sibling_index, dtype=jnp.int32)
    igm, scm, cnt, opn = _sc_call(parent_ig_matrix, parent_sc_matrix, sib16)
    f_out = _tc_f_call(f_input)
    h_out = _tc_h_call(h_input, opn)
    sample_count = cnt[0]
    is_node_open = opn[0].astype(jnp.bool_)
    return (f_out, h_out, igm, scm, sample_count, is_node_open)


# speedup vs baseline: 25.1688x; 1.0333x over previous
import jax
import jax.numpy as jnp
from jax import lax
from jax.experimental import pallas as pl
from jax.experimental.pallas import tpu as pltpu
from jax.experimental.pallas import tpu_sc as plsc

_B = 16384
_LANES = 16
_TILES = 16
_ROWS_PER_TILE = _B // _TILES
_CHUNK = 256


def _sc_body(ig_hbm, sc_hbm, sib_hbm, h_hbm, igm_hbm, scm_hbm, cnt_hbm,
             opn_hbm, ho_hbm,
             ig_v, sc_v, igm_v, scm_v, sib_v, cnt_v, opn_v, h_v, tot_smem):
    cid = lax.axis_index("c")
    sid = lax.axis_index("s")

    @pl.when(cid == 1)
    def _core1():
        hbase = sid * _ROWS_PER_TILE
        for c in range(_ROWS_PER_TILE // _CHUNK):
            hb = hbase + c * _CHUNK
            pltpu.sync_copy(h_hbm.at[pl.ds(hb, _CHUNK)], h_v)
            pltpu.sync_copy(h_v, ho_hbm.at[pl.ds(hb, _CHUNK)])

    @pl.when(cid == 0)
    def _core0():
        base = sid * _ROWS_PER_TILE
        pltpu.sync_copy(sib_hbm, sib_v)
        sib16 = sib_v[...]
        iota16 = lax.iota(jnp.int32, 16)
        acc = jnp.zeros((_LANES,), jnp.int32)

        for c in range(_ROWS_PER_TILE // _CHUNK):
            cb = base + c * _CHUNK
            pltpu.sync_copy(ig_hbm.at[pl.ds(cb, _CHUNK)], ig_v)
            pltpu.sync_copy(sc_hbm.at[pl.ds(cb, _CHUNK)], sc_v)

            def step(j, acc, c=c):
                row = j * _LANES + iota16
                dst = c * _CHUNK + j * _LANES
                igm_v[pl.ds(dst, _LANES)] = plsc.load_gather(
                    ig_v, [row, sib16])
                scx = plsc.load_gather(sc_v, [row, sib16])
                scm_v[pl.ds(dst, _LANES)] = scx
                return acc + scx

            acc = lax.fori_loop(0, _CHUNK // _LANES, step, acc)

        pltpu.sync_copy(igm_v, igm_hbm.at[pl.ds(base, _ROWS_PER_TILE)])
        pltpu.sync_copy(scm_v, scm_hbm.at[pl.ds(base, _ROWS_PER_TILE)])

        my_sum = jnp.sum(acc)

        @pl.when(sid == 0)
        def _init():
            tot_smem[0] = jnp.int32(0)

        plsc.subcore_barrier()
        plsc.fetch_and_add(tot_smem.at[0], my_sum, subcore_id=0)
        plsc.subcore_barrier()

        @pl.when(sid == 0)
        def _finalize():
            total = tot_smem[0].astype(jnp.float32)
            cnt_v[...] = jnp.broadcast_to(total, (_LANES,))
            opn_v[...] = jnp.broadcast_to(
                (total > 0.0).astype(jnp.int32), (_LANES,))
            pltpu.sync_copy(cnt_v, cnt_hbm)
            pltpu.sync_copy(opn_v, opn_hbm)


@jax.jit
def _sc_call(parent_ig_matrix, parent_sc_matrix, sib16, h_input):
    mesh = plsc.VectorSubcoreMesh(core_axis_name="c", subcore_axis_name="s")
    run = pl.kernel(
        _sc_body,
        out_type=[
            jax.ShapeDtypeStruct((_B,), jnp.int32),
            jax.ShapeDtypeStruct((_B,), jnp.int32),
            jax.ShapeDtypeStruct((_LANES,), jnp.float32),
            jax.ShapeDtypeStruct((_LANES,), jnp.int32),
            jax.ShapeDtypeStruct((_B, 128), jnp.float32),
        ],
        mesh=mesh,
        scratch_types=[
            pltpu.VMEM((_CHUNK, 2), jnp.int32),
            pltpu.VMEM((_CHUNK, 2), jnp.int32),
            pltpu.VMEM((_ROWS_PER_TILE,), jnp.int32),
            pltpu.VMEM((_ROWS_PER_TILE,), jnp.int32),
            pltpu.VMEM((_LANES,), jnp.int32),
            pltpu.VMEM((_LANES,), jnp.float32),
            pltpu.VMEM((_LANES,), jnp.int32),
            pltpu.VMEM((_CHUNK, 128), jnp.float32),
            pltpu.SMEM((1,), jnp.int32),
        ],
        compiler_params=pltpu.CompilerParams(
            needs_layout_passes=False,
            use_tc_tiling_on_sc=True,
            skip_device_barrier=True,
        ),
        name="cign_masking_sc",
    )
    return run(parent_ig_matrix, parent_sc_matrix, sib16, h_input)


_N_BLK = 8
_RB = _B // _N_BLK


def _tc_f_body(f_ref, fo_ref):
    fo_ref[...] = f_ref[...]


@jax.jit
def _tc_f_call(f_input):
    return pl.pallas_call(
        _tc_f_body,
        grid=(_N_BLK,),
        in_specs=[pl.BlockSpec((_RB, 1024), lambda i: (i, 0))],
        out_specs=pl.BlockSpec((_RB, 1024), lambda i: (i, 0)),
        out_shape=jax.ShapeDtypeStruct((_B, 1024), jnp.float32),
    )(f_input)


def kernel(f_input, h_input, parent_ig_matrix, parent_sc_matrix, sibling_index):
    sib16 = jnp.full((_LANES,), sibling_index, dtype=jnp.int32)
    igm, scm, cnt, opn, h_out = _sc_call(
        parent_ig_matrix, parent_sc_matrix, sib16, h_input)
    f_out = _tc_f_call(f_input)
    sample_count = cnt[0]
    is_node_open = opn[0].astype(jnp.bool_)
    return (f_out, h_out, igm, scm, sample_count, is_node_open)
